# baseline (speedup 1.0000x reference)
import functools

import jax
import jax.numpy as jnp
from jax import lax
from jax.experimental import pallas as pl
from jax.experimental.pallas import tpu as pltpu
from jax.experimental.pallas import tpu_sc as plsc

P = 1000
CH = 32
_OFFS = [c * 16 for c in range(62)]
_TAIL = 984


def _sc_body(x_hbm, n_hbm, table_hbm, out_hbm, n_vmem, idx_v, rows_v, acc_v, sem):
    half = lax.axis_index("c")
    b = lax.axis_index("s")
    pltpu.sync_copy(n_hbm, n_vmem.at[pl.ds(0, 16)])
    lane = lax.iota(jnp.int32, 16)
    n = n_vmem[pl.ds(b, 16)][0]

    zeros = jnp.zeros((16,), jnp.float32)
    for off in _OFFS:
        acc_v[pl.ds(off, 16)] = zeros
    acc_v[pl.ds(_TAIL, 16)] = zeros

    tail_keep = lane >= 8

    num_chunks = lax.div(n + (CH - 1), CH)
    my_count = lax.div(num_chunks - half + 1, 2)

    def chunk_body(t, carry):
        chunk = half + 2 * t
        start = chunk * CH
        pltpu.sync_copy(x_hbm.at[b, pl.ds(start, CH)], idx_v)
        pltpu.async_copy(table_hbm.at[idx_v], rows_v, sem).wait()
        valid = jnp.minimum(n - start, CH)

        def row_body(j, c2):
            for off in _OFFS:
                plsc.addupdate(acc_v.at[pl.ds(off, 16)], rows_v[j, pl.ds(off, 16)])
            xt = rows_v[j, pl.ds(_TAIL, 16)]
            xt = jnp.where(tail_keep, xt, 0.0)
            plsc.addupdate(acc_v.at[pl.ds(_TAIL, 16)], xt)
            return c2

        lax.fori_loop(0, valid, row_body, 0)
        return carry

    lax.fori_loop(0, my_count, chunk_body, 0)
    pltpu.sync_copy(acc_v, out_hbm.at[half, b])


def _pool_sc(X, N, table):
    mesh = plsc.VectorSubcoreMesh(core_axis_name="c", subcore_axis_name="s")
    f = pl.kernel(
        _sc_body,
        out_type=jax.ShapeDtypeStruct((2, 16, P), jnp.float32),
        mesh=mesh,
        scratch_types=[
            pltpu.VMEM((32,), jnp.int32),
            pltpu.VMEM((CH,), jnp.int32),
            pltpu.VMEM((CH, P), jnp.float32),
            pltpu.VMEM((P,), jnp.float32),
            pltpu.SemaphoreType.DMA,
        ],
        compiler_params=pltpu.CompilerParams(use_tc_tiling_on_sc=False),
    )
    return f(X, N, table)


def _tail_body(part_ref, n_ref, gamma_ref, beta_ref, w_ref, bias_ref, out_ref):
    s = part_ref[0] + part_ref[1]
    nf = n_ref[...].astype(jnp.float32)
    x = s / nf
    mean = jnp.mean(x, axis=1, keepdims=True)
    xc = x - mean
    var = jnp.mean(xc * xc, axis=1, keepdims=True)
    xn = xc * lax.rsqrt(var + 1e-5)
    xn = xn * gamma_ref[...] + beta_ref[...]
    z = jnp.sum(xn * w_ref[...], axis=1, keepdims=True) + bias_ref[...]
    out_ref[...] = jax.nn.sigmoid(z)


def _tail_tc(part, N, gamma, beta, W, b):
    return pl.pallas_call(
        _tail_body,
        out_shape=jax.ShapeDtypeStruct((16, 1), jnp.float32),
    )(part, N.reshape(16, 1), gamma.reshape(1, P), beta.reshape(1, P),
      W.reshape(1, P), b.reshape(1, 1))


@jax.jit
def kernel(X, N, table, gamma, beta, W, b):
    X = X.astype(jnp.int32)
    N = N.astype(jnp.int32)
    part = _pool_sc(X, N, table)
    return _tail_tc(part, N, gamma, beta, W, b).reshape(16)


# speedup vs baseline: 3.8200x; 3.8200x over previous
import functools

import jax
import jax.numpy as jnp
from jax import lax
from jax.experimental import pallas as pl
from jax.experimental.pallas import tpu as pltpu
from jax.experimental.pallas import tpu_sc as plsc

P = 1000
L = 4096
CH = 32
_OFFS = [c * 16 for c in range(62)]
_TAIL = 984


def _sc_body(x_hbm, n_hbm, table_hbm, out_hbm, n_vmem, xrow_v, rows_v, acc_v, sem):
    half = lax.axis_index("c")
    b = lax.axis_index("s")
    pltpu.sync_copy(n_hbm, n_vmem.at[pl.ds(0, 16)])
    n = n_vmem[pl.ds(b, 16)][0]
    pltpu.sync_copy(x_hbm.at[b], xrow_v.at[pl.ds(0, L)])

    zeros = jnp.zeros((16,), jnp.float32)
    for off in _OFFS:
        acc_v[pl.ds(off, 16)] = zeros
    acc_v[pl.ds(_TAIL, 16)] = zeros

    lane = lax.iota(jnp.int32, 16)
    tail_keep = lane >= 8

    num_chunks = lax.div(n + (CH - 1), CH)
    my_count = lax.div(num_chunks - half + 1, 2)

    def chunk_body(t, carry):
        chunk = half + 2 * t
        start = chunk * CH
        valid = jnp.minimum(n - start, CH)

        def issue(j, c2):
            tok = xrow_v[pl.ds(start + j, 16)][0]
            pltpu.async_copy(table_hbm.at[tok], rows_v.at[j], sem)
            return c2

        lax.fori_loop(0, valid, issue, 0)

        def drain(j, c2):
            pltpu.make_async_copy(table_hbm.at[0], rows_v.at[0], sem).wait()
            return c2

        lax.fori_loop(0, valid, drain, 0)

        def row_body(j, c2):
            for off in _OFFS:
                plsc.addupdate(acc_v.at[pl.ds(off, 16)], rows_v[j, pl.ds(off, 16)])
            xt = rows_v[j, pl.ds(_TAIL, 16)]
            xt = jnp.where(tail_keep, xt, 0.0)
            plsc.addupdate(acc_v.at[pl.ds(_TAIL, 16)], xt)
            return c2

        lax.fori_loop(0, valid, row_body, 0)
        return carry

    lax.fori_loop(0, my_count, chunk_body, 0)
    pltpu.sync_copy(acc_v, out_hbm.at[half, b])


def _pool_sc(X, N, table):
    mesh = plsc.VectorSubcoreMesh(core_axis_name="c", subcore_axis_name="s")
    f = pl.kernel(
        _sc_body,
        out_type=jax.ShapeDtypeStruct((2, 16, P), jnp.float32),
        mesh=mesh,
        scratch_types=[
            pltpu.VMEM((32,), jnp.int32),
            pltpu.VMEM((L + 64,), jnp.int32),
            pltpu.VMEM((CH, P), jnp.float32),
            pltpu.VMEM((P,), jnp.float32),
            pltpu.SemaphoreType.DMA,
        ],
    )
    return f(X, N, table)


def _tail_body(part_ref, n_ref, gamma_ref, beta_ref, w_ref, bias_ref, out_ref):
    s = part_ref[0] + part_ref[1]
    nf = n_ref[...].astype(jnp.float32)
    x = s / nf
    mean = jnp.mean(x, axis=1, keepdims=True)
    xc = x - mean
    var = jnp.mean(xc * xc, axis=1, keepdims=True)
    xn = xc * lax.rsqrt(var + 1e-5)
    xn = xn * gamma_ref[...] + beta_ref[...]
    z = jnp.sum(xn * w_ref[...], axis=1, keepdims=True) + bias_ref[...]
    out_ref[...] = jax.nn.sigmoid(z)


def _tail_tc(part, N, gamma, beta, W, b):
    return pl.pallas_call(
        _tail_body,
        out_shape=jax.ShapeDtypeStruct((16, 1), jnp.float32),
    )(part, N.reshape(16, 1), gamma.reshape(1, P), beta.reshape(1, P),
      W.reshape(1, P), b.reshape(1, 1))


@jax.jit
def kernel(X, N, table, gamma, beta, W, b):
    X = X.astype(jnp.int32)
    N = N.astype(jnp.int32)
    part = _pool_sc(X, N, table)
    return _tail_tc(part, N, gamma, beta, W, b).reshape(16)


# speedup vs baseline: 5.0065x; 1.3106x over previous
import functools

import jax
import jax.numpy as jnp
from jax import lax
from jax.experimental import pallas as pl
from jax.experimental.pallas import tpu as pltpu
from jax.experimental.pallas import tpu_sc as plsc

P = 1000
PP = 1008
L = 4096
B = 16
CH = 32
NW = 32
NFULL = P // 16
_OFFS = [c * 16 for c in range(NFULL)]


def _sc_body(x_hbm, n_hbm, table_hbm, out_hbm,
             n_vmem, idx_v, rows0_v, rows1_v, acc_v, stage_v, sem0, sem1):
    cid = lax.axis_index("c")
    sid = lax.axis_index("s")
    w = sid * 2 + cid

    pltpu.sync_copy(n_hbm, n_vmem.at[pl.ds(0, 16)])
    ns = [n_vmem[pl.ds(i, 16)][0] for i in range(B)]
    cum = [jnp.int32(0)]
    for i in range(B):
        cum.append(cum[-1] + lax.div(ns[i] + (CH - 1), CH))
    total = cum[B]

    m = lax.div(jnp.maximum(total - w + (NW - 1), 0), NW)

    def chunk_info(t):
        g = w + NW * t
        b = jnp.int32(0)
        for i in range(1, B):
            b = b + (g >= cum[i]).astype(jnp.int32)
        cb = jnp.int32(0)
        nb = jnp.int32(0)
        for i in range(B):
            is_i = (b == i).astype(jnp.int32)
            cb = cb + is_i * cum[i]
            nb = nb + is_i * ns[i]
        start = (g - cb) * CH
        valid = jnp.minimum(nb - start, CH)
        return b, start, valid

    zeros = jnp.zeros((16,), jnp.float32)

    def zacc(i, c2):
        acc_v[pl.ds(i * 16, 16)] = zeros
        return c2
    lax.fori_loop(0, (B * PP) // 16, zacc, 0)

    lane = lax.iota(jnp.int32, 16)
    tail_keep = lane >= 8

    def issue(t, rows_v, sem):
        b, start, valid = chunk_info(t)
        pltpu.sync_copy(x_hbm.at[b, pl.ds(start, CH)], idx_v.at[pl.ds(0, CH)])

        def one(j, c2):
            tok = idx_v[pl.ds(j, 16)][0]
            pltpu.async_copy(table_hbm.at[tok], rows_v.at[j], sem)
            return c2
        lax.fori_loop(0, valid, one, 0)

    def drain(t, rows_v, sem):
        _, _, valid = chunk_info(t)

        def one(j, c2):
            pltpu.make_async_copy(table_hbm.at[0], rows_v.at[0], sem).wait()
            return c2
        lax.fori_loop(0, valid, one, 0)

    def accum(t, rows_v):
        b, _, valid = chunk_info(t)
        base = b * PP

        def row_body(j, c3):
            for off in _OFFS:
                plsc.addupdate(acc_v.at[pl.ds(base + off, 16)],
                               rows_v[j, pl.ds(off, 16)])
            xt = rows_v[j, pl.ds(P - 16, 16)]
            xt = jnp.where(tail_keep, xt, 0.0)
            plsc.addupdate(acc_v.at[pl.ds(base + P - 16, 16)], xt)
            return c3
        lax.fori_loop(0, valid, row_body, 0)

    @pl.when(m > 0)
    def _prologue():
        issue(0, rows0_v, sem0)

    def pipe(t, c2):
        parity = lax.rem(t, 2)

        @pl.when((t + 1 < m) & (parity == 0))
        def _():
            issue(t + 1, rows1_v, sem1)

        @pl.when((t + 1 < m) & (parity == 1))
        def _():
            issue(t + 1, rows0_v, sem0)

        @pl.when(parity == 0)
        def _():
            drain(t, rows0_v, sem0)
            accum(t, rows0_v)

        @pl.when(parity == 1)
        def _():
            drain(t, rows1_v, sem1)
            accum(t, rows1_v)

        return c2

    lax.fori_loop(0, m, pipe, 0)

    def repack(i, c2):
        for off in range(0, PP, 16):
            stage_v[i, pl.ds(off, 16)] = acc_v[pl.ds(i * PP + off, 16)]
        return c2
    lax.fori_loop(0, B, repack, 0)
    pltpu.sync_copy(stage_v, out_hbm.at[w])


def _pool_sc(X, N, table):
    mesh = plsc.VectorSubcoreMesh(core_axis_name="c", subcore_axis_name="s")
    f = pl.kernel(
        _sc_body,
        out_type=jax.ShapeDtypeStruct((NW, B, PP), jnp.float32),
        mesh=mesh,
        scratch_types=[
            pltpu.VMEM((32,), jnp.int32),
            pltpu.VMEM((CH + 16,), jnp.int32),
            pltpu.VMEM((CH, P), jnp.float32),
            pltpu.VMEM((CH, P), jnp.float32),
            pltpu.VMEM((B * PP,), jnp.float32),
            pltpu.VMEM((B, PP), jnp.float32),
            pltpu.SemaphoreType.DMA,
            pltpu.SemaphoreType.DMA,
        ],
    )
    return f(X, N, table)


def _tail_body(part_ref, n_ref, gamma_ref, beta_ref, w_ref, bias_ref, out_ref):
    s = jnp.sum(part_ref[...], axis=0)[:, :P]
    nf = n_ref[...].astype(jnp.float32)
    x = s / nf
    mean = jnp.mean(x, axis=1, keepdims=True)
    xc = x - mean
    var = jnp.mean(xc * xc, axis=1, keepdims=True)
    xn = xc * lax.rsqrt(var + 1e-5)
    xn = xn * gamma_ref[...] + beta_ref[...]
    z = jnp.sum(xn * w_ref[...], axis=1, keepdims=True) + bias_ref[...]
    out_ref[...] = jax.nn.sigmoid(z)


def _tail_tc(part, N, gamma, beta, W, b):
    return pl.pallas_call(
        _tail_body,
        out_shape=jax.ShapeDtypeStruct((16, 1), jnp.float32),
    )(part, N.reshape(16, 1), gamma.reshape(1, P), beta.reshape(1, P),
      W.reshape(1, P), b.reshape(1, 1))


@jax.jit
def kernel(X, N, table, gamma, beta, W, b):
    X = X.astype(jnp.int32)
    N = N.astype(jnp.int32)
    part = _pool_sc(X, N, table)
    return _tail_tc(part, N, gamma, beta, W, b).reshape(16)


# speedup vs baseline: 12.7976x; 2.5562x over previous
import functools

import jax
import jax.numpy as jnp
from jax import lax
from jax.experimental import pallas as pl
from jax.experimental.pallas import tpu as pltpu
from jax.experimental.pallas import tpu_sc as plsc

P = 1000
L = 4096
B = 16
CH = 32
V = 200019
HALF = 100016
VP = 2 * HALF
SP_STRIDE = HALF
TRASH = B * SP_STRIDE
MAXM = (B * (L // CH) + 15) // 16
K0 = 199680
KTAIL = V - K0


def _sc_body(x_hbm, n_hbm, m_hbm,
             n_vmem, idxstage_v, sidx_v, ones_v, zeros_v, spmem, sem, sem2):
    c = lax.axis_index("c")
    s = lax.axis_index("s")
    lo = c * HALF
    width = HALF

    pltpu.sync_copy(n_hbm, n_vmem.at[pl.ds(0, 16)])
    ns = [n_vmem[pl.ds(i, 16)][0] for i in range(B)]
    cum = [jnp.int32(0)]
    for i in range(B):
        cum.append(cum[-1] + lax.div(ns[i] + (CH - 1), CH))
    total = cum[B]
    m = lax.div(jnp.maximum(total - s + 15, 0), 16)

    def chunk_info(t):
        g = s + 16 * t
        b = jnp.int32(0)
        for i in range(1, B):
            b = b + (g >= cum[i]).astype(jnp.int32)
        cb = jnp.int32(0)
        nb = jnp.int32(0)
        for i in range(B):
            is_i = (b == i).astype(jnp.int32)
            cb = cb + is_i * cum[i]
            nb = nb + is_i * ns[i]
        start = (g - cb) * CH
        valid = jnp.minimum(nb - start, CH)
        return b, start, valid

    def stage(t, c2):
        b, start, _ = chunk_info(t)
        pltpu.async_copy(x_hbm.at[b, pl.ds(start, CH)],
                         idxstage_v.at[t, pl.ds(0, CH)], sem)
        return c2
    lax.fori_loop(0, m, stage, 0)

    zv = jnp.zeros((16,), jnp.float32)

    def zz(i, c2):
        zeros_v[pl.ds(i * 16, 16)] = zv
        return c2
    lax.fori_loop(0, 512, zz, 0)
    for g in range(8):
        ones_v[pl.ds(g * 16, 16)] = jnp.ones((16,), jnp.float32)

    sbase = s * SP_STRIDE
    for i in range(12):
        pltpu.sync_copy(zeros_v, spmem.at[pl.ds(sbase + i * 8192, 8192)])
    pltpu.sync_copy(zeros_v.at[pl.ds(0, 1712)],
                    spmem.at[pl.ds(sbase + 98304, 1712)])
    plsc.subcore_barrier()

    def stage_drain(t, c2):
        pltpu.make_async_copy(x_hbm.at[0, pl.ds(0, CH)],
                              idxstage_v.at[0, pl.ds(0, CH)], sem).wait()
        return c2
    lax.fori_loop(0, m, stage_drain, 0)

    lane = lax.iota(jnp.int32, 16)

    def xform(t, c2):
        b, _, valid = chunk_info(t)
        for g in range(CH // 16):
            tok = idxstage_v[t, pl.ds(g * 16, 16)]
            keep = (tok >= lo) & (tok < lo + width) & ((g * 16 + lane) < valid)
            off = jnp.where(keep, b * SP_STRIDE + tok - lo, TRASH)
            e = t * CH + g * 16
            sidx_v[lax.div(e, 128), 0, pl.ds(lax.rem(e, 128), 16)] = off
        return c2
    lax.fori_loop(0, m, xform, 0)

    tot_e = m * CH
    nstream = lax.div(tot_e + 127, 128)
    trash_vec = jnp.zeros((16,), jnp.int32) + TRASH

    def pad(pg, c2):
        e = tot_e + pg * 16
        sidx_v[lax.div(e, 128), 0, pl.ds(lax.rem(e, 128), 16)] = trash_vec
        return c2
    lax.fori_loop(0, lax.div(nstream * 128 - tot_e, 16), pad, 0)

    def scat(k, c2):
        pltpu.async_copy(ones_v.at[pl.ds(0, 128)],
                         spmem.at[sidx_v.at[k, 0]], sem2, add=True)
        return c2
    lax.fori_loop(0, nstream, scat, 0)

    def scat_drain(k, c2):
        pltpu.make_async_copy(ones_v.at[pl.ds(0, 128)],
                              spmem.at[pl.ds(0, 128)], sem2).wait()
        return c2
    lax.fori_loop(0, nstream, scat_drain, 0)
    plsc.subcore_barrier()

    obase = (s * 2 + c) * HALF
    for i in range(12):
        pltpu.sync_copy(spmem.at[pl.ds(s * SP_STRIDE + i * 8192, 8192)],
                        zeros_v)
        pltpu.sync_copy(zeros_v, m_hbm.at[pl.ds(obase + i * 8192, 8192)])
    pltpu.sync_copy(spmem.at[pl.ds(s * SP_STRIDE + 98304, 1712)],
                    zeros_v.at[pl.ds(0, 1712)])
    pltpu.sync_copy(zeros_v.at[pl.ds(0, 1712)],
                    m_hbm.at[pl.ds(obase + 98304, 1712)])


def _scatter_sc(X, N):
    mesh = plsc.VectorSubcoreMesh(core_axis_name="c", subcore_axis_name="s")
    f = pl.kernel(
        _sc_body,
        out_type=jax.ShapeDtypeStruct((2 * B * HALF,), jnp.float32),
        mesh=mesh,
        scratch_types=[
            pltpu.VMEM((32,), jnp.int32),
            pltpu.VMEM((MAXM, CH + 16), jnp.int32),
            pltpu.VMEM((MAXM * CH // 128, 1, 128), jnp.int32),
            pltpu.VMEM((128,), jnp.float32),
            pltpu.VMEM((8192,), jnp.float32),
            pltpu.VMEM_SHARED((B * SP_STRIDE + 16,), jnp.float32),
            pltpu.SemaphoreType.DMA,
            pltpu.SemaphoreType.DMA,
        ],
    )
    return f(X, N)


def _mm_body(m_ref, t_ref, o_ref):
    @pl.when(pl.program_id(0) == 0)
    def _():
        o_ref[...] = jnp.zeros_like(o_ref)
    o_ref[...] += lax.dot_general(
        m_ref[...], t_ref[...], (((1,), (1,)), ((), ())),
        preferred_element_type=jnp.float32)


def _mm(M16, tableT):
    return pl.pallas_call(
        _mm_body,
        grid=(K0 // 1024,),
        in_specs=[
            pl.BlockSpec((B, 1024), lambda k: (0, k)),
            pl.BlockSpec((P, 1024), lambda k: (0, k)),
        ],
        out_specs=pl.BlockSpec((B, P), lambda k: (0, 0)),
        out_shape=jax.ShapeDtypeStruct((B, P), jnp.float32),
    )(M16, tableT)


def _tail_body(s1_ref, mt_ref, tt_ref, n_ref, gamma_ref, beta_ref,
               w_ref, bias_ref, out_ref):
    s = s1_ref[...] + lax.dot_general(
        mt_ref[...], tt_ref[...], (((1,), (1,)), ((), ())),
        preferred_element_type=jnp.float32)
    nf = n_ref[...].astype(jnp.float32)
    x = s / nf
    mean = jnp.mean(x, axis=1, keepdims=True)
    xc = x - mean
    var = jnp.mean(xc * xc, axis=1, keepdims=True)
    xn = xc * lax.rsqrt(var + 1e-5)
    xn = xn * gamma_ref[...] + beta_ref[...]
    z = jnp.sum(xn * w_ref[...], axis=1, keepdims=True) + bias_ref[...]
    out_ref[...] = jax.nn.sigmoid(z)


def _tail_tc(s1, mt, tt, N, gamma, beta, W, b):
    return pl.pallas_call(
        _tail_body,
        out_shape=jax.ShapeDtypeStruct((B, 1), jnp.float32),
    )(s1, mt, tt, N.reshape(B, 1), gamma.reshape(1, P), beta.reshape(1, P),
      W.reshape(1, P), b.reshape(1, 1))


@jax.jit
def kernel(X, N, table, gamma, beta, W, b):
    X = X.astype(jnp.int32)
    N = N.astype(jnp.int32)
    tableT = table.T
    M16 = _scatter_sc(X, N).reshape(B, VP)
    s1 = _mm(M16, tableT)
    mt = lax.slice(M16, (0, K0), (B, V))
    tt = lax.slice(tableT, (0, K0), (P, V))
    return _tail_tc(s1, mt, tt, N, gamma, beta, W, b).reshape(B)
